# baseline (speedup 1.0000x reference)
import functools

import jax
import jax.numpy as jnp
from jax import lax
from jax.experimental import pallas as pl
from jax.experimental.pallas import tpu as pltpu
from jax.experimental.pallas import tpu_sc as plsc

Q = 1024
D = 128
N = 100000
K = 10
QB = 256
CB = 2000
NQB = Q // QB
NCB = N // CB
LANES = 16
NEG = float("-inf")
BIGI = jnp.iinfo(jnp.int32).max

SC_NC = 2
SC_NS = 16
NW = SC_NC * SC_NS
QPW = Q // NW


def _partial_topk_body(q_ref, c_ref, os_ref, oi_ref):
    b = pl.program_id(1)
    s = lax.dot_general(
        q_ref[...], c_ref[...], (((1,), (1,)), ((), ())),
        preferred_element_type=jnp.float32)
    ids = lax.broadcasted_iota(jnp.int32, (QB, CB), 1) + b * CB

    x, xi = s, ids
    top_s, top_i = [], []
    for _ in range(K):
        m = jnp.max(x, axis=1, keepdims=True)
        aid = jnp.min(jnp.where(x == m, xi, BIGI), axis=1, keepdims=True)
        top_s.append(m)
        top_i.append(aid)
        x = jnp.where(xi == aid, NEG, x)
    ts = jnp.concatenate(
        top_s + [jnp.full((QB, LANES - K), NEG, jnp.float32)], axis=1)
    ti = jnp.concatenate(
        top_i + [jnp.full((QB, LANES - K), BIGI, jnp.int32)], axis=1)
    os_ref[...] = ts.reshape(1, QB, LANES)
    oi_ref[...] = ti.reshape(1, QB, LANES)


def _sc_merge_body(ps_hbm, pi_hbm, os_hbm, oi_hbm, sv, iv, ov, oiv):
    wid = lax.axis_index("s") * SC_NC + lax.axis_index("c")
    base = wid * QPW
    pltpu.sync_copy(ps_hbm.at[:, pl.ds(base, QPW)], sv)
    pltpu.sync_copy(pi_hbm.at[:, pl.ds(base, QPW)], iv)

    def per_query(q, carry):
        run_v = sv[0, q, pl.ds(0, LANES)]
        run_i = iv[0, q, pl.ds(0, LANES)]

        def merge(c, ri):
            rv, rix = ri
            bv = sv[c, q, pl.ds(0, LANES)]
            bi = iv[c, q, pl.ds(0, LANES)]
            bv = lax.rev(bv, (0,))
            bi = lax.rev(bi, (0,))
            take = (rv > bv) | ((rv == bv) & (rix <= bi))
            mv = jnp.where(take, rv, bv)
            mi = jnp.where(take, rix, bi)
            sv2, si2 = plsc.sort_key_val(mv, mi, descending=True)
            return (sv2, si2)

        run_v, run_i = lax.fori_loop(1, NCB, merge, (run_v, run_i))
        ov[q, pl.ds(0, LANES)] = run_v
        oiv[q, pl.ds(0, LANES)] = run_i
        return carry

    lax.fori_loop(0, QPW, per_query, 0)
    pltpu.sync_copy(ov, os_hbm.at[pl.ds(base, QPW)])
    pltpu.sync_copy(oiv, oi_hbm.at[pl.ds(base, QPW)])


def kernel(queries, candidates, identifiers, k):
    assert queries.shape == (Q, D) and candidates.shape == (N, D)
    ps, pi = pl.pallas_call(
        _partial_topk_body,
        grid=(NQB, NCB),
        in_specs=[
            pl.BlockSpec((QB, D), lambda qb, b: (qb, 0)),
            pl.BlockSpec((CB, D), lambda qb, b: (b, 0)),
        ],
        out_specs=[
            pl.BlockSpec((1, QB, LANES), lambda qb, b: (b, qb, 0)),
            pl.BlockSpec((1, QB, LANES), lambda qb, b: (b, qb, 0)),
        ],
        out_shape=[
            jax.ShapeDtypeStruct((NCB, Q, LANES), jnp.float32),
            jax.ShapeDtypeStruct((NCB, Q, LANES), jnp.int32),
        ],
        compiler_params=pltpu.CompilerParams(
            dimension_semantics=("parallel", "arbitrary")),
    )(queries, candidates)

    merge = functools.partial(
        pl.kernel,
        mesh=plsc.VectorSubcoreMesh(core_axis_name="c", subcore_axis_name="s"),
        out_type=[
            jax.ShapeDtypeStruct((Q, LANES), jnp.float32),
            jax.ShapeDtypeStruct((Q, LANES), jnp.int32),
        ],
        scratch_types=[
            pltpu.VMEM((NCB, QPW, LANES), jnp.float32),
            pltpu.VMEM((NCB, QPW, LANES), jnp.int32),
            pltpu.VMEM((QPW, LANES), jnp.float32),
            pltpu.VMEM((QPW, LANES), jnp.int32),
        ],
        compiler_params=pltpu.CompilerParams(
            needs_layout_passes=False, use_tc_tiling_on_sc=False),
    )(_sc_merge_body)
    ts, ti = merge(ps, pi)

    top_ids = jnp.take(identifiers, ti[:, :K], axis=0)
    return ts[:, :K], top_ids


# speedup vs baseline: 1.1542x; 1.1542x over previous
import functools

import jax
import jax.numpy as jnp
from jax import lax
from jax.experimental import pallas as pl
from jax.experimental.pallas import tpu as pltpu
from jax.experimental.pallas import tpu_sc as plsc

Q = 1024
D = 128
N = 100000
K = 10
QB = 256
CB = 5000
NQB = Q // QB
NCB = N // CB
LANES = 16
NEG = float("-inf")
BIGI = jnp.iinfo(jnp.int32).max

SC_NC = 2
SC_NS = 16
NW = SC_NC * SC_NS
QPW = Q // NW


def _partial_topk_body(q_ref, c_ref, os_ref, oi_ref):
    b = pl.program_id(1)
    s = lax.dot_general(
        q_ref[...], c_ref[...], (((1,), (1,)), ((), ())),
        preferred_element_type=jnp.float32)
    ids = lax.broadcasted_iota(jnp.int32, (QB, CB), 1) + b * CB

    x, xi = s, ids
    top_s, top_i = [], []
    for _ in range(K):
        m = jnp.max(x, axis=1, keepdims=True)
        aid = jnp.min(jnp.where(x == m, xi, BIGI), axis=1, keepdims=True)
        top_s.append(m)
        top_i.append(aid)
        x = jnp.where(xi == aid, NEG, x)
    ts = jnp.concatenate(
        top_s + [jnp.full((QB, LANES - K), NEG, jnp.float32)], axis=1)
    ti = jnp.concatenate(
        top_i + [jnp.full((QB, LANES - K), BIGI, jnp.int32)], axis=1)
    os_ref[...] = ts.reshape(1, QB, LANES)
    oi_ref[...] = ti.reshape(1, QB, LANES)


def _sc_merge_body(ps_hbm, pi_hbm, os_hbm, oi_hbm, sv, iv, ov, oiv):
    wid = lax.axis_index("s") * SC_NC + lax.axis_index("c")
    base = wid * QPW
    pltpu.sync_copy(ps_hbm.at[:, pl.ds(base, QPW)], sv)
    pltpu.sync_copy(pi_hbm.at[:, pl.ds(base, QPW)], iv)

    def per_query(q, carry):
        run_v = sv[0, q, pl.ds(0, LANES)]
        run_i = iv[0, q, pl.ds(0, LANES)]

        def merge(c, ri):
            rv, rix = ri
            bv = sv[c, q, pl.ds(0, LANES)]
            bi = iv[c, q, pl.ds(0, LANES)]
            bv = lax.rev(bv, (0,))
            bi = lax.rev(bi, (0,))
            take = (rv > bv) | ((rv == bv) & (rix <= bi))
            mv = jnp.where(take, rv, bv)
            mi = jnp.where(take, rix, bi)
            sv2, si2 = plsc.sort_key_val(mv, mi, descending=True)
            return (sv2, si2)

        run_v, run_i = lax.fori_loop(1, NCB, merge, (run_v, run_i))
        ov[q, pl.ds(0, LANES)] = run_v
        oiv[q, pl.ds(0, LANES)] = run_i
        return carry

    lax.fori_loop(0, QPW, per_query, 0)
    pltpu.sync_copy(ov, os_hbm.at[pl.ds(base, QPW)])
    pltpu.sync_copy(oiv, oi_hbm.at[pl.ds(base, QPW)])


def kernel(queries, candidates, identifiers, k):
    assert queries.shape == (Q, D) and candidates.shape == (N, D)
    ps, pi = pl.pallas_call(
        _partial_topk_body,
        grid=(NQB, NCB),
        in_specs=[
            pl.BlockSpec((QB, D), lambda qb, b: (qb, 0)),
            pl.BlockSpec((CB, D), lambda qb, b: (b, 0)),
        ],
        out_specs=[
            pl.BlockSpec((1, QB, LANES), lambda qb, b: (b, qb, 0)),
            pl.BlockSpec((1, QB, LANES), lambda qb, b: (b, qb, 0)),
        ],
        out_shape=[
            jax.ShapeDtypeStruct((NCB, Q, LANES), jnp.float32),
            jax.ShapeDtypeStruct((NCB, Q, LANES), jnp.int32),
        ],
        compiler_params=pltpu.CompilerParams(
            dimension_semantics=("parallel", "arbitrary")),
    )(queries, candidates)

    merge = functools.partial(
        pl.kernel,
        mesh=plsc.VectorSubcoreMesh(core_axis_name="c", subcore_axis_name="s"),
        out_type=[
            jax.ShapeDtypeStruct((Q, LANES), jnp.float32),
            jax.ShapeDtypeStruct((Q, LANES), jnp.int32),
        ],
        scratch_types=[
            pltpu.VMEM((NCB, QPW, LANES), jnp.float32),
            pltpu.VMEM((NCB, QPW, LANES), jnp.int32),
            pltpu.VMEM((QPW, LANES), jnp.float32),
            pltpu.VMEM((QPW, LANES), jnp.int32),
        ],
        compiler_params=pltpu.CompilerParams(
            needs_layout_passes=False, use_tc_tiling_on_sc=False),
    )(_sc_merge_body)
    ts, ti = merge(ps, pi)

    top_ids = jnp.take(identifiers, ti[:, :K], axis=0)
    return ts[:, :K], top_ids


# speedup vs baseline: 1.2281x; 1.0640x over previous
import functools

import jax
import jax.numpy as jnp
from jax import lax
from jax.experimental import pallas as pl
from jax.experimental.pallas import tpu as pltpu
from jax.experimental.pallas import tpu_sc as plsc

Q = 1024
D = 128
N = 100000
K = 10
QB = 256
CB = 10000
NQB = Q // QB
NCB = N // CB
LANES = 16
NEG = float("-inf")
BIGI = jnp.iinfo(jnp.int32).max

SC_NC = 2
SC_NS = 16
NW = SC_NC * SC_NS
QPW = Q // NW


def _partial_topk_body(q_ref, c_ref, os_ref, oi_ref):
    b = pl.program_id(1)
    s = lax.dot_general(
        q_ref[...], c_ref[...], (((1,), (1,)), ((), ())),
        preferred_element_type=jnp.float32)
    ids = lax.broadcasted_iota(jnp.int32, (QB, CB), 1) + b * CB

    x, xi = s, ids
    top_s, top_i = [], []
    for _ in range(K):
        m = jnp.max(x, axis=1, keepdims=True)
        aid = jnp.min(jnp.where(x == m, xi, BIGI), axis=1, keepdims=True)
        top_s.append(m)
        top_i.append(aid)
        x = jnp.where(xi == aid, NEG, x)
    ts = jnp.concatenate(
        top_s + [jnp.full((QB, LANES - K), NEG, jnp.float32)], axis=1)
    ti = jnp.concatenate(
        top_i + [jnp.full((QB, LANES - K), BIGI, jnp.int32)], axis=1)
    os_ref[...] = ts.reshape(1, QB, LANES)
    oi_ref[...] = ti.reshape(1, QB, LANES)


def _sc_merge_body(ps_hbm, pi_hbm, os_hbm, oi_hbm, sv, iv, ov, oiv):
    wid = lax.axis_index("s") * SC_NC + lax.axis_index("c")
    base = wid * QPW
    pltpu.sync_copy(ps_hbm.at[:, pl.ds(base, QPW)], sv)
    pltpu.sync_copy(pi_hbm.at[:, pl.ds(base, QPW)], iv)

    def per_query(q, carry):
        run_v = sv[0, q, pl.ds(0, LANES)]
        run_i = iv[0, q, pl.ds(0, LANES)]

        def merge(c, ri):
            rv, rix = ri
            bv = sv[c, q, pl.ds(0, LANES)]
            bi = iv[c, q, pl.ds(0, LANES)]
            bv = lax.rev(bv, (0,))
            bi = lax.rev(bi, (0,))
            take = (rv > bv) | ((rv == bv) & (rix <= bi))
            mv = jnp.where(take, rv, bv)
            mi = jnp.where(take, rix, bi)
            sv2, si2 = plsc.sort_key_val(mv, mi, descending=True)
            return (sv2, si2)

        run_v, run_i = lax.fori_loop(1, NCB, merge, (run_v, run_i))
        ov[q, pl.ds(0, LANES)] = run_v
        oiv[q, pl.ds(0, LANES)] = run_i
        return carry

    lax.fori_loop(0, QPW, per_query, 0)
    pltpu.sync_copy(ov, os_hbm.at[pl.ds(base, QPW)])
    pltpu.sync_copy(oiv, oi_hbm.at[pl.ds(base, QPW)])


def kernel(queries, candidates, identifiers, k):
    assert queries.shape == (Q, D) and candidates.shape == (N, D)
    ps, pi = pl.pallas_call(
        _partial_topk_body,
        grid=(NQB, NCB),
        in_specs=[
            pl.BlockSpec((QB, D), lambda qb, b: (qb, 0)),
            pl.BlockSpec((CB, D), lambda qb, b: (b, 0)),
        ],
        out_specs=[
            pl.BlockSpec((1, QB, LANES), lambda qb, b: (b, qb, 0)),
            pl.BlockSpec((1, QB, LANES), lambda qb, b: (b, qb, 0)),
        ],
        out_shape=[
            jax.ShapeDtypeStruct((NCB, Q, LANES), jnp.float32),
            jax.ShapeDtypeStruct((NCB, Q, LANES), jnp.int32),
        ],
        compiler_params=pltpu.CompilerParams(
            dimension_semantics=("parallel", "arbitrary")),
    )(queries, candidates)

    merge = functools.partial(
        pl.kernel,
        mesh=plsc.VectorSubcoreMesh(core_axis_name="c", subcore_axis_name="s"),
        out_type=[
            jax.ShapeDtypeStruct((Q, LANES), jnp.float32),
            jax.ShapeDtypeStruct((Q, LANES), jnp.int32),
        ],
        scratch_types=[
            pltpu.VMEM((NCB, QPW, LANES), jnp.float32),
            pltpu.VMEM((NCB, QPW, LANES), jnp.int32),
            pltpu.VMEM((QPW, LANES), jnp.float32),
            pltpu.VMEM((QPW, LANES), jnp.int32),
        ],
        compiler_params=pltpu.CompilerParams(
            needs_layout_passes=False, use_tc_tiling_on_sc=False),
    )(_sc_merge_body)
    ts, ti = merge(ps, pi)

    top_ids = jnp.take(identifiers, ti[:, :K], axis=0)
    return ts[:, :K], top_ids
